# baseline (speedup 1.0000x reference)
import functools

import jax
import jax.numpy as jnp
from jax import lax
from jax.experimental import pallas as pl
from jax.experimental.pallas import tpu as pltpu
from jax.experimental.pallas import tpu_sc as plsc

_NC = 2
_NS = 16
_NW = _NC * _NS
_L = 16
_CH = 32


@functools.lru_cache(maxsize=None)
def _make_gather(B: int, V: int, D: int):
    assert B % (8 * _NW) == 0
    bpw = B // _NW
    D2 = 2 * D

    mesh = plsc.VectorSubcoreMesh(core_axis_name="c", subcore_axis_name="s")

    @functools.partial(
        pl.kernel,
        out_type=jax.ShapeDtypeStruct((B // 2, D2), jnp.float32),
        mesh=mesh,
        scratch_types=[
            pltpu.VMEM((bpw,), jnp.int32),
            pltpu.VMEM((2, _CH, 8, D), jnp.float32),
            pltpu.VMEM((_CH // 2, D2), jnp.float32),
            pltpu.SemaphoreType.DMA,
            pltpu.SemaphoreType.DMA,
        ],
        compiler_params=pltpu.CompilerParams(
            use_tc_tiling_on_sc=True, needs_layout_passes=False),
    )
    def gather(table_hbm, idx_hbm, out_hbm, ids_v, buf_v, out_v, semA, semB):
        wid = lax.axis_index("s") * _NC + lax.axis_index("c")
        base = pl.multiple_of(wid * bpw, bpw)
        pltpu.sync_copy(idx_hbm.at[pl.ds(base, bpw)], ids_v)
        lanes = lax.iota(jnp.int32, _L)
        sems = (semA, semB)
        n_waves = bpw // _CH

        def fire_wave(c, slot):
            sem = sems[slot]

            def fire(k, _, c=c, slot=slot, sem=sem):
                v16 = ids_v[pl.ds(c * _CH + k * _L, _L)]
                for l in range(_L):
                    t = jnp.max(jnp.where(lanes == l, v16, 0))
                    r8 = pl.multiple_of(
                        jax.lax.shift_left(
                            jax.lax.shift_right_logical(t, 3), 3), 8)
                    pltpu.make_async_copy(
                        table_hbm.at[pl.ds(r8, 8), :],
                        buf_v.at[slot, k * _L + l],
                        sem,
                    ).start()
                return _
            lax.fori_loop(0, _CH // _L, fire, 0)

        def drain_wave(slot):
            sem = sems[slot]

            def drain(j, _, slot=slot, sem=sem):
                pltpu.make_async_copy(
                    table_hbm.at[pl.ds(0, 8), :], buf_v.at[slot, 0], sem,
                ).wait()
                return _
            lax.fori_loop(0, _CH, drain, 0, unroll=8)

        def extract_wave(c, slot):
            def extract(k, _, c=c, slot=slot):
                j = lax.iota(jnp.int32, _L) + k * _L
                ids16 = ids_v[pl.ds(c * _CH + k * _L, _L)]
                rlo = jax.lax.bitwise_and(ids16, jnp.int32(7))
                jp = jax.lax.shift_right_logical(j, 1)
                poff = jax.lax.shift_left(
                    jax.lax.bitwise_and(j, jnp.int32(1)), jnp.int32(6))
                sv = jnp.full((_L,), slot, jnp.int32)

                def inner(f, _2):
                    fv = jnp.full((_L,), 0, jnp.int32) + f
                    vals = plsc.load_gather(buf_v, [sv, j, rlo, fv])
                    plsc.store_scatter(out_v, [jp, poff + f], vals)
                    return _2
                lax.fori_loop(0, D, inner, 0, unroll=8)
                return _
            lax.fori_loop(0, _CH // _L, extract, 0)

            orow = pl.multiple_of((base + c * _CH) // 2, _CH // 2)
            pltpu.sync_copy(out_v, out_hbm.at[pl.ds(orow, _CH // 2), :])

        n_pairs = n_waves // 2
        fire_wave(0, 0)

        def pair(p, _):
            c0 = p * 2
            fire_wave(c0 + 1, 1)
            drain_wave(0)
            extract_wave(c0, 0)

            @pl.when(p + 1 < n_pairs)
            def _fire_next():
                fire_wave(c0 + 2, 0)

            drain_wave(1)
            extract_wave(c0 + 1, 1)
            return _
        lax.fori_loop(0, n_pairs, pair, 0)

    return gather


def kernel(input_ids, labels, alibi, attention_mask, embed_table):
    V, D = embed_table.shape
    ids = input_ids.reshape(-1).astype(jnp.int32)
    B = ids.shape[0]
    hidden = _make_gather(B, V, D)(embed_table, ids)
    hidden = hidden.reshape(input_ids.shape + (D,))
    return (hidden, labels, alibi, attention_mask)


# speedup vs baseline: 1.0019x; 1.0019x over previous
import functools

import jax
import jax.numpy as jnp
from jax import lax
from jax.experimental import pallas as pl
from jax.experimental.pallas import tpu as pltpu
from jax.experimental.pallas import tpu_sc as plsc

_NC = 2
_NS = 16
_NW = _NC * _NS
_L = 16
_CH = 32


@functools.lru_cache(maxsize=None)
def _make_gather(B: int, V: int, D: int):
    assert B % (8 * _NW) == 0
    bpw = B // _NW
    D2 = 2 * D

    mesh = plsc.VectorSubcoreMesh(core_axis_name="c", subcore_axis_name="s")

    @functools.partial(
        pl.kernel,
        out_type=jax.ShapeDtypeStruct((B // 2, D2), jnp.float32),
        mesh=mesh,
        scratch_types=[
            pltpu.VMEM((bpw,), jnp.int32),
            pltpu.VMEM((2, _CH, 8, D), jnp.float32),
            pltpu.VMEM((_CH // 2, D2), jnp.float32),
            pltpu.SemaphoreType.DMA,
            pltpu.SemaphoreType.DMA,
        ],
        compiler_params=pltpu.CompilerParams(
            use_tc_tiling_on_sc=True, needs_layout_passes=False),
    )
    def gather(table_hbm, idx_hbm, out_hbm, ids_v, buf_v, out_v, semA, semB):
        wid = lax.axis_index("s") * _NC + lax.axis_index("c")
        base = pl.multiple_of(wid * bpw, bpw)
        pltpu.sync_copy(idx_hbm.at[pl.ds(base, bpw)], ids_v)
        lanes = lax.iota(jnp.int32, _L)
        sems = (semA, semB)
        n_waves = bpw // _CH

        def fire_wave(c, slot):
            sem = sems[slot]

            def fire(k, _, c=c, slot=slot, sem=sem):
                v16 = ids_v[pl.ds(c * _CH + k * _L, _L)]
                for l in range(_L):
                    t = jnp.max(jnp.where(lanes == l, v16, 0))
                    r8 = pl.multiple_of(
                        jax.lax.shift_left(
                            jax.lax.shift_right_logical(t, 3), 3), 8)
                    pltpu.make_async_copy(
                        table_hbm.at[pl.ds(r8, 8), :],
                        buf_v.at[slot, k * _L + l],
                        sem,
                    ).start()
                return _
            lax.fori_loop(0, _CH // _L, fire, 0)

        def drain_wave(slot):
            sem = sems[slot]

            def drain(j, _, slot=slot, sem=sem):
                pltpu.make_async_copy(
                    table_hbm.at[pl.ds(0, 8), :], buf_v.at[slot, 0], sem,
                ).wait()
                return _
            lax.fori_loop(0, _CH, drain, 0, unroll=8)

        def extract_wave(c, slot):
            def extract(k, _, c=c, slot=slot):
                j = lax.iota(jnp.int32, _L) + k * _L
                ids16 = ids_v[pl.ds(c * _CH + k * _L, _L)]
                rlo = jax.lax.bitwise_and(ids16, jnp.int32(7))
                jp = jax.lax.shift_right_logical(j, 1)
                poff = jax.lax.shift_left(
                    jax.lax.bitwise_and(j, jnp.int32(1)), jnp.int32(6))
                sv = jnp.full((_L,), slot, jnp.int32)

                def inner(f, _2):
                    fv = jnp.full((_L,), 0, jnp.int32) + f
                    vals = plsc.load_gather(buf_v, [sv, j, rlo, fv])
                    plsc.store_scatter(out_v, [jp, poff + f], vals)
                    return _2
                lax.fori_loop(0, D, inner, 0, unroll=8)
                return _
            lax.fori_loop(0, _CH // _L, extract, 0)

            orow = pl.multiple_of((base + c * _CH) // 2, _CH // 2)
            pltpu.sync_copy(out_v, out_hbm.at[pl.ds(orow, _CH // 2), :])

        n_pairs = n_waves // 2
        fire_wave(0, 0)

        def pair(p, _):
            c0 = p * 2
            fire_wave(c0 + 1, 1)
            drain_wave(0)
            extract_wave(c0, 0)

            @pl.when(p + 1 < n_pairs)
            def _fire_next():
                fire_wave(c0 + 2, 0)

            drain_wave(1)
            extract_wave(c0 + 1, 1)
            return _
        lax.fori_loop(0, n_pairs, pair, 0)

    return gather


def kernel(input_ids, labels, alibi, attention_mask, embed_table):
    V, D = embed_table.shape
    ids = input_ids.reshape(-1).astype(jnp.int32)
    B = ids.shape[0]
    hidden = _make_gather(B, V, D)(embed_table, ids)
    hidden = hidden.reshape(input_ids.shape + (D,))
    one = (ids[0] >= 0).astype(alibi.dtype)
    alibi_out = alibi * one
    return (hidden, labels, alibi_out, attention_mask)


# speedup vs baseline: 1.0056x; 1.0037x over previous
import functools

import jax
import jax.numpy as jnp
from jax import lax
from jax.experimental import pallas as pl
from jax.experimental.pallas import tpu as pltpu
from jax.experimental.pallas import tpu_sc as plsc

_NC = 2
_NS = 16
_NW = _NC * _NS
_L = 16
_CH = 32


@functools.lru_cache(maxsize=None)
def _make_gather(B: int, V: int, D: int):
    assert B % (8 * _NW) == 0
    bpw = B // _NW
    D2 = 2 * D

    mesh = plsc.VectorSubcoreMesh(core_axis_name="c", subcore_axis_name="s")

    @functools.partial(
        pl.kernel,
        out_type=jax.ShapeDtypeStruct((B // 2, D2), jnp.float32),
        mesh=mesh,
        scratch_types=[
            pltpu.VMEM((bpw,), jnp.int32),
            pltpu.VMEM((2, _CH, 8, D), jnp.float32),
            pltpu.VMEM((_CH // 2, D2), jnp.float32),
            pltpu.SemaphoreType.DMA,
            pltpu.SemaphoreType.DMA,
        ],
        compiler_params=pltpu.CompilerParams(
            use_tc_tiling_on_sc=True, needs_layout_passes=False),
    )
    def gather(table_hbm, idx_hbm, out_hbm, ids_v, buf_v, out_v, semA, semB):
        wid = lax.axis_index("s") * _NC + lax.axis_index("c")
        base = pl.multiple_of(wid * bpw, bpw)
        pltpu.sync_copy(idx_hbm.at[pl.ds(base, bpw)], ids_v)
        lanes = lax.iota(jnp.int32, _L)
        sems = (semA, semB)
        n_waves = bpw // _CH

        def fire_wave(c, slot):
            sem = sems[slot]

            def fire(k, _, c=c, slot=slot, sem=sem):
                v16 = ids_v[pl.ds(c * _CH + k * _L, _L)]
                for l in range(_L):
                    t = jnp.max(jnp.where(lanes == l, v16, 0))
                    r8 = pl.multiple_of(
                        jax.lax.shift_left(
                            jax.lax.shift_right_logical(t, 3), 3), 8)
                    pltpu.make_async_copy(
                        table_hbm.at[pl.ds(r8, 8), :],
                        buf_v.at[slot, k * _L + l],
                        sem,
                    ).start()
                return _
            lax.fori_loop(0, _CH // _L, fire, 0)

        def drain_wave(slot):
            sem = sems[slot]

            def drain(j, _, slot=slot, sem=sem):
                pltpu.make_async_copy(
                    table_hbm.at[pl.ds(0, 8), :], buf_v.at[slot, 0], sem,
                ).wait()
                return _
            lax.fori_loop(0, _CH, drain, 0, unroll=8)

        def extract_wave(c, slot):
            def extract(k, _, c=c, slot=slot):
                j = lax.iota(jnp.int32, _L) + k * _L
                ids16 = ids_v[pl.ds(c * _CH + k * _L, _L)]
                rlo = jax.lax.bitwise_and(ids16, jnp.int32(7))
                jp = jax.lax.shift_right_logical(j, 1)
                poff = jax.lax.shift_left(
                    jax.lax.bitwise_and(j, jnp.int32(1)), jnp.int32(6))
                sv = jnp.full((_L,), slot, jnp.int32)

                def inner(f, _2):
                    fv = jnp.full((_L,), 0, jnp.int32) + f
                    vals = plsc.load_gather(buf_v, [sv, j, rlo, fv])
                    plsc.store_scatter(out_v, [jp, poff + f], vals)
                    return _2
                lax.fori_loop(0, D, inner, 0, unroll=8)
                return _
            lax.fori_loop(0, _CH // _L, extract, 0)

            orow = pl.multiple_of((base + c * _CH) // 2, _CH // 2)
            pltpu.sync_copy(out_v, out_hbm.at[pl.ds(orow, _CH // 2), :])

        n_pairs = n_waves // 2
        fire_wave(0, 0)

        def pair(p, _):
            c0 = p * 2
            fire_wave(c0 + 1, 1)
            drain_wave(0)
            extract_wave(c0, 0)

            @pl.when(p + 1 < n_pairs)
            def _fire_next():
                fire_wave(c0 + 2, 0)

            drain_wave(1)
            extract_wave(c0 + 1, 1)
            return _
        lax.fori_loop(0, n_pairs, pair, 0)

    return gather


def kernel(input_ids, labels, alibi, attention_mask, embed_table):
    V, D = embed_table.shape
    ids = input_ids.reshape(-1).astype(jnp.int32)
    B = ids.shape[0]
    hidden = _make_gather(B, V, D)(embed_table, ids)
    hidden = hidden.reshape(input_ids.shape + (D,))
    return (hidden, labels, alibi, attention_mask)
